# baseline (speedup 1.0000x reference)
import functools

import jax
import jax.numpy as jnp
from jax import lax
from jax.experimental import pallas as pl
from jax.experimental.pallas import tpu as pltpu
from jax.experimental.pallas import tpu_sc as plsc

N = 10000
E = 320000
D = 128
H = 128
P = 8192

NC = 2
NS = 16
NW = NC * NS
CH = 128
NCH = 80
EW = NCH * CH
EP = NW * EW
NPAD = 10240
RPT = NPAD // NS
NB = 2
PH = 2
CPP = NCH // PH

PW = (2 * P) // NW
PCH = PW // CH

_MESH = dict(core_axis_name="c", subcore_axis_name="s")


def _make_agg(with_deg):
    out_type = [jax.ShapeDtypeStruct((NC, NPAD, D), jnp.float32)]
    if with_deg:
        out_type.append(jax.ShapeDtypeStruct((NC, NPAD), jnp.float32))
    scratch = (
        [pltpu.VMEM((CPP, CH), jnp.int32),
         pltpu.VMEM((CPP, CH), jnp.int32)]
        + [pltpu.VMEM((CH, D), jnp.float32) for _ in range(NB)]
        + [pltpu.VMEM((CH,), jnp.float32),
           pltpu.VMEM_SHARED((NPAD, D), jnp.float32),
           pltpu.VMEM_SHARED((NPAD,), jnp.float32)]
        + [pltpu.SemaphoreType.DMA for _ in range(NB + 1)]
    )

    @functools.partial(
        pl.kernel,
        mesh=plsc.VectorSubcoreMesh(**_MESH),
        out_type=out_type,
        scratch_types=scratch,
    )
    def agg_kernel(table, srcp, dstp, zeros2, zeros1, ones_h, *rest):
        if with_deg:
            agg_out, deg_out = rest[0], rest[1]
            rest = rest[2:]
        else:
            agg_out = rest[0]
            rest = rest[1:]
        src_v, dst_v = rest[0], rest[1]
        bufs = rest[2:2 + NB]
        ones_v, acc_sh, deg_sh = rest[2 + NB:5 + NB]
        sems = rest[5 + NB:5 + 2 * NB]
        sem_d = rest[5 + 2 * NB]
        c = lax.axis_index("c")
        s = lax.axis_index("s")
        wid = c * NS + s

        pltpu.sync_copy(zeros2, acc_sh.at[pl.ds(s * RPT, RPT)])
        if with_deg:
            pltpu.sync_copy(zeros1, deg_sh.at[pl.ds(s * RPT, RPT)])
            pltpu.sync_copy(ones_h, ones_v)
        plsc.subcore_barrier()

        for p in range(PH):
            pltpu.sync_copy(srcp.at[wid, pl.ds(p * CPP, CPP)], src_v)
            pltpu.sync_copy(dstp.at[wid, pl.ds(p * CPP, CPP)], dst_v)
            for b in range(NB - 1):
                pltpu.async_copy(table.at[src_v.at[b]], bufs[b], sems[b])

            def body(i, carry):
                jj = i * NB
                for b in range(NB):
                    j = jj + b
                    nx = j + NB - 1
                    nb = (b + NB - 1) % NB

                    @pl.when(nx < CPP)
                    def _():
                        pltpu.async_copy(table.at[src_v.at[nx]], bufs[nb],
                                         sems[nb])

                    pltpu.make_async_copy(table.at[src_v.at[j]], bufs[b],
                                          sems[b]).wait()
                    if with_deg:
                        @pl.when(j > 0)
                        def _():
                            pltpu.make_async_copy(
                                ones_v, deg_sh.at[dst_v.at[j - 1]],
                                sem_d).wait()
                        pltpu.async_copy(ones_v, deg_sh.at[dst_v.at[j]],
                                         sem_d, add=True)
                    pltpu.sync_copy(bufs[b], acc_sh.at[dst_v.at[j]], add=True)
                return carry

            lax.fori_loop(0, CPP // NB, body, 0)
            if with_deg:
                pltpu.make_async_copy(
                    ones_v, deg_sh.at[dst_v.at[CPP - 1]], sem_d).wait()
        plsc.subcore_barrier()

        pltpu.sync_copy(acc_sh.at[pl.ds(s * RPT, RPT)],
                        agg_out.at[c, pl.ds(s * RPT, RPT)])
        if with_deg:
            pltpu.sync_copy(deg_sh.at[pl.ds(s * RPT, RPT)],
                            deg_out.at[c, pl.ds(s * RPT, RPT)])

    return agg_kernel


_agg = _make_agg(True)


@functools.partial(
    pl.kernel,
    mesh=plsc.VectorSubcoreMesh(**_MESH),
    out_type=jax.ShapeDtypeStruct((2 * P, D), jnp.float32),
    scratch_types=[
        pltpu.VMEM((PCH, CH), jnp.int32),
        pltpu.VMEM((CH, D), jnp.float32),
        pltpu.SemaphoreType.DMA,
    ],
)
def _pair_gather(table, idxp, out, idx_v, rows_v, sem):
    c = lax.axis_index("c")
    s = lax.axis_index("s")
    wid = c * NS + s
    pltpu.sync_copy(idxp.at[wid], idx_v)

    def body(j, carry):
        pltpu.async_copy(table.at[idx_v.at[j]], rows_v, sem).wait()
        pltpu.sync_copy(rows_v, out.at[pl.ds(wid * PW + j * CH, CH)])
        return carry

    lax.fori_loop(0, PCH, body, 0)


def _emb_body(x_ref, w_ref, b_ref, o_ref):
    o_ref[...] = lax.dot_general(
        x_ref[...], w_ref[...], (((1,), (1,)), ((), ())),
        preferred_element_type=jnp.float32) + b_ref[...]


def _conv_body(act, aggp_ref, degp_ref, h_ref, wl_ref, bl_ref, wr_ref, o_ref):
    agg = aggp_ref[0, :N] + aggp_ref[1, :N]
    deg = jnp.maximum(degp_ref[0, :N] + degp_ref[1, :N], 1.0)
    agg = agg / deg
    o = lax.dot_general(agg, wl_ref[...], (((1,), (1,)), ((), ())),
                        preferred_element_type=jnp.float32) + bl_ref[...]
    o = o + lax.dot_general(h_ref[...], wr_ref[...], (((1,), (1,)), ((), ())),
                            preferred_element_type=jnp.float32)
    if act:
        o = jnp.where(o > 0, o, 0.1 * o)
    o_ref[...] = o


def _head_body(hp_ref, w1_ref, b1_ref, w2_ref, b2_ref, o_ref):
    u = lax.dot_general(hp_ref[...], w1_ref[...], (((1,), (1,)), ((), ())),
                        preferred_element_type=jnp.float32) + b1_ref[...]
    u = jnp.where(u > 0, u, 0.1 * u)
    o_ref[...] = lax.dot_general(
        u, w2_ref[...], (((1,), (1,)), ((), ())),
        preferred_element_type=jnp.float32) + b2_ref[...]


def _emb(x, w, b):
    return pl.pallas_call(
        _emb_body,
        out_shape=jax.ShapeDtypeStruct((N, D), jnp.float32),
    )(x, w, b.reshape(1, H))


def _conv(act, aggp, degp, h, wl, bl, wr):
    return pl.pallas_call(
        functools.partial(_conv_body, act),
        out_shape=jax.ShapeDtypeStruct((N, H), jnp.float32),
    )(aggp, degp, h, wl, bl.reshape(1, H), wr)


def _head(hp, w1, b1, w2, b2):
    w2r = jnp.broadcast_to(w2.reshape(1, H), (H, H))
    b2r = jnp.broadcast_to(b2.reshape(1, 1), (1, H))
    o = pl.pallas_call(
        _head_body,
        out_shape=jax.ShapeDtypeStruct((P, H), jnp.float32),
    )(hp, w1, b1.reshape(1, H), w2r, b2r)
    return o[:, :1]


def kernel(x, edge_index, edge_attr, pairs, W_emb, b_emb, Wl0, bl0, Wr0,
           Wl1, bl1, Wr1, W1, b1, W2, b2):
    src = edge_index[0]
    dst = edge_index[1]
    srcp = jnp.pad(src, (0, EP - E)).reshape(NW, NCH, CH)
    dstp = jnp.pad(dst, (0, EP - E), constant_values=N).reshape(NW, NCH, CH)
    zeros2 = jnp.zeros((RPT, D), jnp.float32)
    zeros1 = jnp.zeros((RPT,), jnp.float32)
    ones_h = jnp.ones((CH,), jnp.float32)
    idxp = pairs.reshape(NW, PCH, CH)

    h0 = _emb(x, W_emb, b_emb)
    aggp0, degp = _agg(h0, srcp, dstp, zeros2, zeros1, ones_h)
    deg = degp[:, :, None]
    h1 = _conv(True, aggp0, deg, h0, Wl0, bl0, Wr0)
    aggp1, _ = _agg(h1, srcp, dstp, zeros2, zeros1, ones_h)
    h2 = _conv(False, aggp1, deg, h1, Wl1, bl1, Wr1)
    rows = _pair_gather(h2, idxp)
    hp = rows.reshape(P, 2 * H)
    return _head(hp, W1, b1, W2, b2)
